# baseline (device time: 12349 ns/iter reference)
import jax
import jax.numpy as jnp
from jax import lax
from jax.experimental import pallas as pl
from jax.experimental.pallas import tpu as pltpu

N_DEV = 8
N_CHUNKS = 8


def kernel(x):
    m_per, n = x.shape
    assert m_per % N_CHUNKS == 0
    chunk = m_per // N_CHUNKS
    x = pltpu.with_memory_space_constraint(x, pltpu.MemorySpace.HBM)

    def body(x_hbm, out_ref, xv_ref, acc_ref, comm_ref, in_sems, send_sems,
             recv_sems):
        my = lax.axis_index("i")

        barrier_sem = pltpu.get_barrier_semaphore()
        for j in range(1, N_DEV):
            pl.semaphore_signal(
                barrier_sem,
                inc=1,
                device_id=(lax.rem(my + j, N_DEV),),
                device_id_type=pl.DeviceIdType.MESH,
            )

        cps = [
            pltpu.make_async_copy(
                x_hbm.at[pl.ds(c * chunk, chunk), :],
                xv_ref.at[c % 2],
                in_sems.at[c % 2],
            )
            for c in range(N_CHUNKS)
        ]
        cps[0].start()
        cps[1].start()
        for c in range(N_CHUNKS):
            cps[c].wait()
            part = jnp.sum(xv_ref[c % 2], axis=0, keepdims=True)
            if c + 2 < N_CHUNKS:
                cps[c + 2].start()
            if c == 0:
                acc_ref[...] = part
            else:
                acc_ref[...] = acc_ref[...] + part

        pl.semaphore_wait(barrier_sem, N_DEV - 1)

        rdmas = []
        for j in range(1, N_DEV):
            dst = lax.rem(my + j, N_DEV)
            rdma = pltpu.make_async_remote_copy(
                src_ref=acc_ref,
                dst_ref=comm_ref.at[N_DEV - 1 - j],
                send_sem=send_sems.at[j - 1],
                recv_sem=recv_sems.at[N_DEV - 1 - j],
                device_id=(dst,),
                device_id_type=pl.DeviceIdType.MESH,
            )
            rdma.start()
            rdmas.append(rdma)
        for rdma in rdmas:
            rdma.wait()

        out_ref[...] = acc_ref[...] + jnp.sum(comm_ref[...], axis=0)

    return pl.pallas_call(
        body,
        out_shape=jax.ShapeDtypeStruct((1, n), x.dtype),
        in_specs=[pl.BlockSpec(memory_space=pl.ANY)],
        out_specs=pl.BlockSpec(memory_space=pltpu.VMEM),
        scratch_shapes=[
            pltpu.VMEM((2, chunk, n), x.dtype),
            pltpu.VMEM((1, n), x.dtype),
            pltpu.VMEM((N_DEV - 1, 1, n), x.dtype),
            pltpu.SemaphoreType.DMA((2,)),
            pltpu.SemaphoreType.DMA((N_DEV - 1,)),
            pltpu.SemaphoreType.DMA((N_DEV - 1,)),
        ],
        compiler_params=pltpu.CompilerParams(collective_id=0),
    )(x)
